# Initial kernel scaffold; baseline (speedup 1.0000x reference)
import jax
import jax.numpy as jnp
from jax.experimental import pallas as pl


def kernel(memory_addresses, pointer_ids):
    raise NotImplementedError("write your pallas kernel here")


# speedup vs baseline: 69.7376x; 69.7376x over previous
import functools

import jax
import jax.numpy as jnp
from jax import lax
from jax.experimental import pallas as pl
from jax.experimental.pallas import tpu as pltpu
from jax.experimental.pallas import tpu_sc as plsc

_NC = 2
_NS = 16
_NW = _NC * _NS

_B = 4096
_BPW = _B // _NW


def _worker_base():
    wid = lax.axis_index("c") * _NS + lax.axis_index("s")
    return wid * _BPW


def _make_malloc_kernel():
    @functools.partial(
        pl.kernel,
        mesh=plsc.VectorSubcoreMesh(core_axis_name="c", subcore_axis_name="s"),
        out_type=[jax.ShapeDtypeStruct((_B,), jnp.int32)],
        scratch_types=[
            pltpu.VMEM((_BPW,), jnp.int32),
            pltpu.VMEM((_BPW,), jnp.int32),
            pltpu.SemaphoreType.DMA,
        ],
    )
    def k(ptr_hbm, slot_hbm, inv_hbm, idx_v, pos_v, sem):
        base = _worker_base()
        pltpu.sync_copy(ptr_hbm.at[pl.ds(base, _BPW)], idx_v)
        pltpu.sync_copy(slot_hbm.at[pl.ds(base, _BPW)], pos_v)
        pltpu.async_copy(pos_v, inv_hbm.at[idx_v], sem).wait()

    return k


def _make_deref_kernel():
    @functools.partial(
        pl.kernel,
        mesh=plsc.VectorSubcoreMesh(core_axis_name="c", subcore_axis_name="s"),
        out_type=[
            jax.ShapeDtypeStruct((_B,), jnp.int32),
            jax.ShapeDtypeStruct((_B,), jnp.int32),
        ],
        scratch_types=[
            pltpu.VMEM((_BPW,), jnp.int32),
            pltpu.VMEM((_BPW,), jnp.int32),
            pltpu.VMEM((_BPW,), jnp.int32),
            pltpu.VMEM((_BPW,), jnp.int32),
            pltpu.SemaphoreType.DMA,
        ],
    )
    def k(ptr_hbm, inv_hbm, lo_hbm, hi_hbm, out_lo, out_hi,
          idx_v, chosen_v, lo_v, hi_v, sem):
        base = _worker_base()
        pltpu.sync_copy(ptr_hbm.at[pl.ds(base, _BPW)], idx_v)
        pltpu.async_copy(inv_hbm.at[idx_v], chosen_v, sem).wait()
        pltpu.async_copy(lo_hbm.at[chosen_v], lo_v, sem).wait()
        pltpu.async_copy(hi_hbm.at[chosen_v], hi_v, sem).wait()
        pltpu.sync_copy(lo_v, out_lo.at[pl.ds(base, _BPW)])
        pltpu.sync_copy(hi_v, out_hi.at[pl.ds(base, _BPW)])

    return k


_malloc_kernel = _make_malloc_kernel()
_deref_kernel = _make_deref_kernel()


def kernel(memory_addresses, pointer_ids):
    parts = lax.bitcast_convert_type(memory_addresses, jnp.int32)
    lo = parts[..., 0]
    hi = parts[..., 1]
    ptr = pointer_ids.astype(jnp.int32)
    slots = jnp.arange(_B, dtype=jnp.int32)
    (inv,) = _malloc_kernel(ptr, slots)
    out_lo, out_hi = _deref_kernel(ptr, inv, lo, hi)
    pair = jnp.stack([out_lo, out_hi], axis=-1)
    return lax.bitcast_convert_type(pair, jnp.int64)


# speedup vs baseline: 166.8030x; 2.3919x over previous
import functools

import jax
import jax.numpy as jnp
from jax import lax
from jax.experimental import pallas as pl
from jax.experimental.pallas import tpu as pltpu
from jax.experimental.pallas import tpu_sc as plsc

_NC = 2
_NS = 16
_NW = _NC * _NS

_B = 4096
_BPW = _B // _NW


def _make_sc_kernel():
    @functools.partial(
        pl.kernel,
        mesh=plsc.VectorSubcoreMesh(core_axis_name="c", subcore_axis_name="s"),
        out_type=[
            jax.ShapeDtypeStruct((_B,), jnp.int32),
            jax.ShapeDtypeStruct((_B,), jnp.int32),
        ],
        scratch_types=[
            pltpu.VMEM_SHARED((_B,), jnp.int32),
            pltpu.VMEM((_BPW,), jnp.int32),
            pltpu.VMEM((_BPW,), jnp.int32),
            pltpu.VMEM((_BPW,), jnp.int32),
            pltpu.VMEM((_BPW,), jnp.int32),
            pltpu.VMEM((_BPW,), jnp.int32),
            pltpu.SemaphoreType.DMA,
        ],
    )
    def k(ptr_hbm, slot_hbm, lo_hbm, hi_hbm, out_lo, out_hi,
          inv_s, idx_v, pos_v, chosen_v, lo_v, hi_v, sem):
        wid = lax.axis_index("c") * _NS + lax.axis_index("s")
        base = wid * _BPW
        pltpu.sync_copy(ptr_hbm.at[pl.ds(base, _BPW)], idx_v)
        pltpu.sync_copy(slot_hbm.at[pl.ds(base, _BPW)], pos_v)
        pltpu.async_copy(pos_v, inv_s.at[idx_v], sem).wait()
        plsc.subcore_barrier()
        pltpu.async_copy(inv_s.at[idx_v], chosen_v, sem).wait()
        pltpu.async_copy(lo_hbm.at[chosen_v], lo_v, sem).wait()
        pltpu.async_copy(hi_hbm.at[chosen_v], hi_v, sem).wait()
        pltpu.sync_copy(lo_v, out_lo.at[pl.ds(base, _BPW)])
        pltpu.sync_copy(hi_v, out_hi.at[pl.ds(base, _BPW)])

    return k


_sc_kernel = _make_sc_kernel()


def kernel(memory_addresses, pointer_ids):
    parts = lax.bitcast_convert_type(memory_addresses, jnp.int32)
    lo = parts[..., 0]
    hi = parts[..., 1]
    ptr = pointer_ids.astype(jnp.int32)
    slots = jnp.arange(_B, dtype=jnp.int32)
    out_lo, out_hi = _sc_kernel(ptr, slots, lo, hi)
    pair = jnp.stack([out_lo, out_hi], axis=-1)
    return lax.bitcast_convert_type(pair, jnp.int64)
